# Initial kernel scaffold; baseline (speedup 1.0000x reference)
import jax
import jax.numpy as jnp
from jax.experimental import pallas as pl


def kernel(x, W1, W2):
    raise NotImplementedError("write your pallas kernel here")


# speedup vs baseline: 26.4599x; 26.4599x over previous
import functools

import jax
import jax.numpy as jnp
import numpy as np
from jax.experimental import pallas as pl
from jax.experimental.pallas import tpu as pltpu

VOCAB = 10
DIM = 10
TOK = 200
LANES = TOK * DIM
ROWS = 16384
BR = 512


def _expand_matrix():
    e = np.zeros((TOK, LANES), dtype=np.float32)
    for t in range(TOK):
        e[t, DIM * t:DIM * (t + 1)] = 1.0
    return jnp.asarray(e, dtype=jnp.bfloat16)


def _body(x_ref, w1_ref, w2_ref, e_ref, out_ref):
    x = x_ref[...].astype(jnp.bfloat16)
    xe = jnp.dot(x, e_ref[...], preferred_element_type=jnp.float32)

    wsum = w1_ref[...] + w2_ref[...]
    lane = jax.lax.broadcasted_iota(jnp.int32, (1, LANES), 1)
    d = lane % DIM
    rows = jnp.zeros((VOCAB, LANES), jnp.float32)
    for dd in range(DIM):
        rows = jnp.where(d == dd, wsum[:, dd:dd + 1], rows)

    out = jnp.zeros((BR, LANES), jnp.float32)
    for v in range(VOCAB):
        out = jnp.where(xe == float(v), rows[v:v + 1, :], out)
    out_ref[...] = out


@jax.jit
def kernel(x, W1, W2):
    e = _expand_matrix()
    grid = (ROWS // BR,)
    out2d = pl.pallas_call(
        _body,
        grid=grid,
        in_specs=[
            pl.BlockSpec((BR, TOK), lambda i: (i, 0)),
            pl.BlockSpec((VOCAB, DIM), lambda i: (0, 0)),
            pl.BlockSpec((VOCAB, DIM), lambda i: (0, 0)),
            pl.BlockSpec((TOK, LANES), lambda i: (0, 0)),
        ],
        out_specs=pl.BlockSpec((BR, LANES), lambda i: (i, 0)),
        out_shape=jax.ShapeDtypeStruct((ROWS, LANES), jnp.float32),
    )(x, W1, W2, e)
    return out2d.reshape(ROWS, TOK, DIM)
